# baseline (speedup 1.0000x reference)
import functools

import jax
import jax.numpy as jnp
from jax.experimental import pallas as pl
from jax.experimental.pallas import tpu as pltpu


def _body(ha_ref, hb_ref, he_ref, x_ref, wm_ref, bm_ref,
          gamma_ref, beta_ref, w1_ref, b1_ref, a1_ref, w2_ref, b2_ref,
          wl_ref, bl_ref,
          enc_ref, xp_ref, loss_ref,
          g1_ref, g2_ref, lsum_ref, *, nb, denom):
    p = pl.program_id(0)
    i = pl.program_id(1)
    br = ha_ref.shape[0]

    @pl.when((p == 0) & (i == 0))
    def _():
        h = jnp.dot(he_ref[...], wm_ref[...],
                    preferred_element_type=jnp.float32)
        h = h + bm_ref[...]
        h = jnp.where(h > 0, h, 0.1 * h)
        mean = jnp.mean(h, axis=0, keepdims=True)
        var = jnp.mean((h - mean) ** 2, axis=0, keepdims=True)
        h = (h - mean) / jnp.sqrt(var + 1e-5) * gamma_ref[...] + beta_ref[...]
        g1_ref[...] = jnp.dot(h, w1_ref[...],
                              preferred_element_type=jnp.float32
                              ).astype(jnp.bfloat16)

    @pl.when(p == 0)
    def _():
        a1 = a1_ref[0, 0]
        g_bf = g1_ref[...]
        for j, h_ref in enumerate((ha_ref, hb_ref)):
            y = jnp.dot(h_ref[...].astype(jnp.bfloat16), g_bf,
                        preferred_element_type=jnp.float32)
            y = y + b1_ref[...]
            x1 = jnp.where(y > 0, y, a1 * y)
            g2_ref[pl.ds((2 * i + j) * br, br), :] = jnp.dot(
                x1, w2_ref[...], preferred_element_type=jnp.float32)

    @pl.when(p == 1)
    def _():
        g_bf = g2_ref[...].astype(jnp.bfloat16)
        psum = jnp.float32(0.0)
        for j, h_ref in enumerate((ha_ref, hb_ref)):
            enc = jnp.dot(h_ref[...].astype(jnp.bfloat16), g_bf,
                          preferred_element_type=jnp.float32)
            enc = enc + b2_ref[...]
            enc_ref[pl.ds(j * br, br), :] = enc
            z = jnp.where(enc > 0, enc, 0.01 * enc)
            xp = jnp.dot(z, wl_ref[...], preferred_element_type=jnp.float32)
            xp = xp + bl_ref[...]
            xp_ref[pl.ds(j * br, br), :] = xp
            dd = xp - x_ref[pl.ds(j * br, br), :]
            psum = psum + jnp.sum(dd * dd)
        prev = jnp.where(i == 0, 0.0, lsum_ref[0])
        tot = prev + psum
        lsum_ref[0] = tot

        @pl.when(i == nb - 1)
        def _():
            loss_ref[...] = jnp.reshape(tot / denom, (1, 1))


def kernel(H, he_rep, x, W_mlp, b_mlp, gamma, beta, W1, b1, a1, W2, b2,
           W_lin, b_lin):
    n, d = he_rep.shape
    br = 200
    nb = n // (2 * br)

    def rep(_p, _i):
        return (0, 0)

    ha = pl.BlockSpec((br, n), lambda p, i: (2 * i, 0))
    hb = pl.BlockSpec((br, n), lambda p, i: (2 * i + 1, 0))
    row2 = pl.BlockSpec((2 * br, d), lambda p, i: (p * i, 0))

    enc, x_prime, loss_arr = pl.pallas_call(
        functools.partial(_body, nb=nb, denom=float(n * d)),
        grid=(2, nb),
        in_specs=[
            ha, hb,
            pl.BlockSpec((n, d), rep),
            row2,
            pl.BlockSpec((d, d), rep),
            pl.BlockSpec((1, d), rep),
            pl.BlockSpec((1, d), rep),
            pl.BlockSpec((1, d), rep),
            pl.BlockSpec((d, d), rep),
            pl.BlockSpec((1, d), rep),
            pl.BlockSpec((1, 1), rep),
            pl.BlockSpec((d, d), rep),
            pl.BlockSpec((1, d), rep),
            pl.BlockSpec((d, d), rep),
            pl.BlockSpec((1, d), rep),
        ],
        out_specs=[row2, row2, pl.BlockSpec((1, 1), rep)],
        out_shape=[
            jax.ShapeDtypeStruct((n, d), jnp.float32),
            jax.ShapeDtypeStruct((n, d), jnp.float32),
            jax.ShapeDtypeStruct((1, 1), jnp.float32),
        ],
        scratch_shapes=[
            pltpu.VMEM((n, d), jnp.bfloat16),
            pltpu.VMEM((n, d), jnp.float32),
            pltpu.SMEM((1,), jnp.float32),
        ],
        compiler_params=pltpu.CompilerParams(
            vmem_limit_bytes=63 * 1024 * 1024),
    )(H, H, he_rep, x, W_mlp, b_mlp.reshape(1, d), gamma.reshape(1, d),
      beta.reshape(1, d), W1, b1.reshape(1, d), a1.reshape(1, 1), W2,
      b2.reshape(1, d), W_lin, b_lin.reshape(1, d))

    return (loss_arr[0, 0], x_prime, enc)


# speedup vs baseline: 1.0048x; 1.0048x over previous
import functools

import jax
import jax.numpy as jnp
from jax.experimental import pallas as pl
from jax.experimental.pallas import tpu as pltpu


def _body(ha_ref, hb_ref, he_ref, x_ref, wm_ref, bm_ref,
          gamma_ref, beta_ref, w1_ref, b1_ref, a1_ref, w2_ref, b2_ref,
          wl_ref, bl_ref,
          enc_ref, xp_ref, loss_ref,
          g1_ref, g2_ref, lsum_ref, *, nb, denom):
    p = pl.program_id(0)
    i = pl.program_id(1)
    br = ha_ref.shape[0]

    @pl.when((p == 0) & (i == 0))
    def _():
        h = jnp.dot(he_ref[...], wm_ref[...],
                    preferred_element_type=jnp.float32)
        h = h + bm_ref[...]
        h = jnp.where(h > 0, h, 0.1 * h)
        mean = jnp.mean(h, axis=0, keepdims=True)
        var = jnp.mean((h - mean) ** 2, axis=0, keepdims=True)
        h = (h - mean) / jnp.sqrt(var + 1e-5) * gamma_ref[...] + beta_ref[...]
        g1_ref[...] = jnp.dot(h, w1_ref[...],
                              preferred_element_type=jnp.float32
                              ).astype(jnp.bfloat16)

    @pl.when(p == 0)
    def _():
        a1 = a1_ref[0, 0]
        g_bf = g1_ref[...]
        halves = []
        for h_ref in (ha_ref, hb_ref):
            y = jnp.dot(h_ref[...].astype(jnp.bfloat16), g_bf,
                        preferred_element_type=jnp.float32)
            y = y + b1_ref[...]
            x1 = jnp.where(y > 0, y, a1 * y)
            halves.append(jnp.dot(
                x1, w2_ref[...],
                preferred_element_type=jnp.float32).astype(jnp.bfloat16))
        g2_ref[pl.ds(i * 2 * br, 2 * br), :] = jnp.concatenate(halves, axis=0)

    @pl.when(p == 1)
    def _():
        g_bf = g2_ref[...]
        psum = jnp.float32(0.0)
        for j, h_ref in enumerate((ha_ref, hb_ref)):
            enc = jnp.dot(h_ref[...].astype(jnp.bfloat16), g_bf,
                          preferred_element_type=jnp.float32)
            enc = enc + b2_ref[...]
            enc_ref[pl.ds(j * br, br), :] = enc
            z = jnp.where(enc > 0, enc, 0.01 * enc)
            xp = jnp.dot(z, wl_ref[...], preferred_element_type=jnp.float32)
            xp = xp + bl_ref[...]
            xp_ref[pl.ds(j * br, br), :] = xp
            dd = xp - x_ref[pl.ds(j * br, br), :]
            psum = psum + jnp.sum(dd * dd)
        prev = jnp.where(i == 0, 0.0, lsum_ref[0])
        tot = prev + psum
        lsum_ref[0] = tot

        @pl.when(i == nb - 1)
        def _():
            loss_ref[...] = jnp.reshape(tot / denom, (1, 1))


def kernel(H, he_rep, x, W_mlp, b_mlp, gamma, beta, W1, b1, a1, W2, b2,
           W_lin, b_lin):
    n, d = he_rep.shape
    br = 200
    nb = n // (2 * br)

    def rep(_p, _i):
        return (0, 0)

    ha = pl.BlockSpec((br, n), lambda p, i: (2 * i, 0))
    hb = pl.BlockSpec((br, n), lambda p, i: (2 * i + 1, 0))
    row2 = pl.BlockSpec((2 * br, d), lambda p, i: (p * i, 0))

    enc, x_prime, loss_arr = pl.pallas_call(
        functools.partial(_body, nb=nb, denom=float(n * d)),
        grid=(2, nb),
        in_specs=[
            ha, hb,
            pl.BlockSpec((n, d), rep),
            row2,
            pl.BlockSpec((d, d), rep),
            pl.BlockSpec((1, d), rep),
            pl.BlockSpec((1, d), rep),
            pl.BlockSpec((1, d), rep),
            pl.BlockSpec((d, d), rep),
            pl.BlockSpec((1, d), rep),
            pl.BlockSpec((1, 1), rep),
            pl.BlockSpec((d, d), rep),
            pl.BlockSpec((1, d), rep),
            pl.BlockSpec((d, d), rep),
            pl.BlockSpec((1, d), rep),
        ],
        out_specs=[row2, row2, pl.BlockSpec((1, 1), rep)],
        out_shape=[
            jax.ShapeDtypeStruct((n, d), jnp.float32),
            jax.ShapeDtypeStruct((n, d), jnp.float32),
            jax.ShapeDtypeStruct((1, 1), jnp.float32),
        ],
        scratch_shapes=[
            pltpu.VMEM((n, d), jnp.bfloat16),
            pltpu.VMEM((n, d), jnp.bfloat16),
            pltpu.SMEM((1,), jnp.float32),
        ],
        compiler_params=pltpu.CompilerParams(
            vmem_limit_bytes=63 * 1024 * 1024),
    )(H, H, he_rep, x, W_mlp, b_mlp.reshape(1, d), gamma.reshape(1, d),
      beta.reshape(1, d), W1, b1.reshape(1, d), a1.reshape(1, 1), W2,
      b2.reshape(1, d), W_lin, b_lin.reshape(1, d))

    return (loss_arr[0, 0], x_prime, enc)
